# baseline (speedup 1.0000x reference)
import functools

import jax
import jax.numpy as jnp
from jax import lax
from jax.experimental import pallas as pl
from jax.experimental.pallas import tpu as pltpu
from jax.experimental.pallas import tpu_sc as plsc

N_NODES = 10000
N_EDGES = 320000
D_FEAT = 128
TIME_WINDOW = 500

_B = 128
_NCHUNK = N_EDGES // _B
_TILES = 32
_GMAX = -(-_NCHUNK // _TILES)
_NROWS = 10240
_ZROWS = _NROWS // 16


def _sc_body(x_hbm, src_hbm, dst_hbm, et_hbm, st_hbm, p_out, c_out,
             acc, accc, srcv, dstv, etv, stv, deff, rows, onesv, zb2, zb1,
             sem):
    cid = lax.axis_index("c")
    sid = lax.axis_index("s")
    wid = sid * 2 + cid

    z16 = jnp.zeros((16,), jnp.float32)
    for i in range(16):
        for j in range(8):
            zb2[i, pl.ds(j * 16, 16)] = z16
    for k in range(_ZROWS // 16):
        zb1[pl.ds(k * 16, 16)] = z16
    for j in range(8):
        onesv[pl.ds(j * 16, 16)] = jnp.ones((16,), jnp.float32)

    def zloop(k, carry):
        pltpu.sync_copy(zb2, acc.at[pl.ds(sid * _ZROWS + k * 16, 16)])
        return carry

    lax.fori_loop(0, _ZROWS // 16, zloop, None)
    pltpu.sync_copy(zb1, accc.at[pl.ds(sid * _ZROWS, _ZROWS)])

    plsc.subcore_barrier()

    def chunk(g, carry):
        c = g * _TILES + wid
        off = c * _B
        pltpu.sync_copy(src_hbm.at[pl.ds(off, _B)], srcv)
        pltpu.sync_copy(dst_hbm.at[pl.ds(off, _B)], dstv)
        pltpu.sync_copy(et_hbm.at[pl.ds(off, _B)], etv)
        pltpu.async_copy(st_hbm.at[dstv], stv, sem).wait()
        dummy = N_NODES + sid * 15 + lax.rem(lax.iota(jnp.int32, 16),
                                             jnp.full((16,), 15, jnp.int32))
        for j in range(_B // 16):
            sl = pl.ds(j * 16, 16)
            et = etv[sl]
            st = stv[sl]
            m = (et <= st) & (et > st - TIME_WINDOW)
            deff[sl] = jnp.where(m, dstv[sl], dummy)
        pltpu.async_copy(x_hbm.at[srcv], rows, sem).wait()
        pltpu.sync_copy(rows, acc.at[deff], add=True)
        pltpu.sync_copy(onesv, accc.at[deff], add=True)
        return carry

    lax.fori_loop(0, 80, chunk, None)

    plsc.subcore_barrier()

    pltpu.sync_copy(acc.at[pl.ds(sid * _ZROWS, _ZROWS)],
                    p_out.at[pl.ds(cid * _NROWS + sid * _ZROWS, _ZROWS)])
    pltpu.sync_copy(accc.at[pl.ds(sid * _ZROWS, _ZROWS)],
                    c_out.at[pl.ds(cid * _NROWS + sid * _ZROWS, _ZROWS)])


_sc_call = functools.partial(
    pl.kernel,
    out_type=[
        jax.ShapeDtypeStruct((2 * _NROWS, D_FEAT), jnp.float32),
        jax.ShapeDtypeStruct((2 * _NROWS,), jnp.float32),
    ],
    mesh=plsc.VectorSubcoreMesh(core_axis_name="c", subcore_axis_name="s"),
    scratch_types=[
        pltpu.VMEM_SHARED((_NROWS, D_FEAT), jnp.float32),
        pltpu.VMEM_SHARED((_NROWS,), jnp.float32),
        pltpu.VMEM((_B,), jnp.int32),
        pltpu.VMEM((_B,), jnp.int32),
        pltpu.VMEM((_B,), jnp.int32),
        pltpu.VMEM((_B,), jnp.int32),
        pltpu.VMEM((_B,), jnp.int32),
        pltpu.VMEM((_B, D_FEAT), jnp.float32),
        pltpu.VMEM((_B,), jnp.float32),
        pltpu.VMEM((16, D_FEAT), jnp.float32),
        pltpu.VMEM((_ZROWS,), jnp.float32),
        pltpu.SemaphoreType.DMA,
    ],
)(_sc_body)


def _combine_body(x_ref, p0_ref, p1_ref, c0_ref, c1_ref, o_ref):
    cnt = c0_ref[0, 0, :] + c1_ref[0, 0, :]
    s = p0_ref[...] + p1_ref[...]
    o_ref[...] = x_ref[...] + s / jnp.clip(cnt, 1.0, None)[:, None]


_R = 1000


def _combine(x, p0, p1, c0, c1):
    return pl.pallas_call(
        _combine_body,
        grid=(N_NODES // _R,),
        in_specs=[
            pl.BlockSpec((_R, D_FEAT), lambda i: (i, 0)),
            pl.BlockSpec((_R, D_FEAT), lambda i: (i, 0)),
            pl.BlockSpec((_R, D_FEAT), lambda i: (i, 0)),
            pl.BlockSpec((1, 1, _R), lambda i: (i, 0, 0)),
            pl.BlockSpec((1, 1, _R), lambda i: (i, 0, 0)),
        ],
        out_specs=pl.BlockSpec((_R, D_FEAT), lambda i: (i, 0)),
        out_shape=jax.ShapeDtypeStruct((N_NODES, D_FEAT), jnp.float32),
    )(x, p0, p1, c0, c1)


@jax.jit
def kernel(x, edge_index, edge_time, seed_time):
    pad = 11776
    src = jnp.concatenate([edge_index[0], jnp.zeros((pad,), jnp.int32)])
    dst = jnp.concatenate([edge_index[1], jnp.zeros((pad,), jnp.int32)])
    et = jnp.concatenate([edge_time, jnp.full((pad,), 2 ** 30, jnp.int32)])
    pr, cr = _sc_call(x, src, dst, et, seed_time)
    p0 = pr[:N_NODES]
    p1 = pr[_NROWS:_NROWS + N_NODES]
    c0 = cr[:N_NODES].reshape(N_NODES // _R, 1, _R)
    c1 = cr[_NROWS:_NROWS + N_NODES].reshape(N_NODES // _R, 1, _R)
    return _combine(x, p0, p1, c0, c1)


# speedup vs baseline: 1.6559x; 1.6559x over previous
import functools

import jax
import jax.numpy as jnp
from jax import lax
from jax.experimental import pallas as pl
from jax.experimental.pallas import tpu as pltpu
from jax.experimental.pallas import tpu_sc as plsc

N_NODES = 10000
N_EDGES = 320000
D_FEAT = 128
TIME_WINDOW = 500

_B = 128
_NCHUNK = N_EDGES // _B
_TILES = 32
_GMAX = -(-_NCHUNK // _TILES)
_NROWS = 10240
_ZROWS = _NROWS // 16


def _sc_body(x_hbm, src_hbm, dst_hbm, et_hbm, st_hbm, p_out, c_out,
             acc, accc, srcv, dstv, etv, stv, deff, rows, onesv, zb2, zb1,
             sem):
    cid = lax.axis_index("c")
    sid = lax.axis_index("s")
    wid = sid * 2 + cid

    z16 = jnp.zeros((16,), jnp.float32)
    for i in range(16):
        for j in range(8):
            zb2[i, pl.ds(j * 16, 16)] = z16
    for k in range(_ZROWS // 16):
        zb1[pl.ds(k * 16, 16)] = z16
    for j in range(8):
        onesv[pl.ds(j * 16, 16)] = jnp.ones((16,), jnp.float32)

    def zloop(k, carry):
        pltpu.sync_copy(zb2, acc.at[pl.ds(sid * _ZROWS + k * 16, 16)])
        return carry

    lax.fori_loop(0, _ZROWS // 16, zloop, None)
    pltpu.sync_copy(zb1, accc.at[pl.ds(sid * _ZROWS, _ZROWS)])

    plsc.subcore_barrier()

    def chunk(g, carry):
        c = g * _TILES + wid
        off = c * _B
        pltpu.sync_copy(src_hbm.at[pl.ds(off, _B)], srcv)
        pltpu.sync_copy(dst_hbm.at[pl.ds(off, _B)], dstv)
        pltpu.sync_copy(et_hbm.at[pl.ds(off, _B)], etv)
        pltpu.async_copy(st_hbm.at[dstv], stv, sem).wait()
        dummy = N_NODES + sid * 15 + lax.rem(lax.iota(jnp.int32, 16),
                                             jnp.full((16,), 15, jnp.int32))
        for j in range(_B // 16):
            sl = pl.ds(j * 16, 16)
            et = etv[sl]
            st = stv[sl]
            m = (et <= st) & (et > st - TIME_WINDOW)
            deff[sl] = jnp.where(m, dstv[sl], dummy)
        pltpu.async_copy(x_hbm.at[srcv], rows, sem).wait()
        pltpu.sync_copy(rows, acc.at[deff], add=True)
        pltpu.sync_copy(onesv, accc.at[deff], add=True)
        return carry

    lax.fori_loop(0, 80, chunk, None)

    plsc.subcore_barrier()

    pltpu.sync_copy(acc.at[pl.ds(sid * _ZROWS, _ZROWS)],
                    p_out.at[pl.ds(cid * _NROWS + sid * _ZROWS, _ZROWS)])
    pltpu.sync_copy(accc.at[pl.ds(sid * _ZROWS, _ZROWS)],
                    c_out.at[pl.ds(cid * _NROWS + sid * _ZROWS, _ZROWS)])


_sc_call = functools.partial(
    pl.kernel,
    out_type=[
        jax.ShapeDtypeStruct((2 * _NROWS, D_FEAT), jnp.float32),
        jax.ShapeDtypeStruct((2 * _NROWS,), jnp.float32),
    ],
    mesh=plsc.VectorSubcoreMesh(core_axis_name="c", subcore_axis_name="s"),
    scratch_types=[
        pltpu.VMEM_SHARED((_NROWS, D_FEAT), jnp.float32),
        pltpu.VMEM_SHARED((_NROWS,), jnp.float32),
        pltpu.VMEM((_B,), jnp.int32),
        pltpu.VMEM((_B,), jnp.int32),
        pltpu.VMEM((_B,), jnp.int32),
        pltpu.VMEM((_B,), jnp.int32),
        pltpu.VMEM((_B,), jnp.int32),
        pltpu.VMEM((_B, D_FEAT), jnp.float32),
        pltpu.VMEM((_B,), jnp.float32),
        pltpu.VMEM((16, D_FEAT), jnp.float32),
        pltpu.VMEM((_ZROWS,), jnp.float32),
        pltpu.SemaphoreType.DMA,
    ],
)(_sc_body)


def _combine_body(x_ref, p0_ref, p1_ref, c0_ref, c1_ref, o_ref):
    cnt = c0_ref[0, 0, :] + c1_ref[0, 0, :]
    s = p0_ref[...] + p1_ref[...]
    o_ref[...] = x_ref[...] + s / jnp.clip(cnt, 1.0, None)[:, None]


_R = 1000


def _combine(x, p0, p1, c0, c1):
    return pl.pallas_call(
        _combine_body,
        grid=(N_NODES // _R,),
        in_specs=[
            pl.BlockSpec((_R, D_FEAT), lambda i: (i, 0)),
            pl.BlockSpec((_R, D_FEAT), lambda i: (i, 0)),
            pl.BlockSpec((_R, D_FEAT), lambda i: (i, 0)),
            pl.BlockSpec((1, 1, _R), lambda i: (i, 0, 0)),
            pl.BlockSpec((1, 1, _R), lambda i: (i, 0, 0)),
        ],
        out_specs=pl.BlockSpec((_R, D_FEAT), lambda i: (i, 0)),
        out_shape=jax.ShapeDtypeStruct((N_NODES, D_FEAT), jnp.float32),
    )(x, p0, p1, c0, c1)


@jax.jit
def kernel(x, edge_index, edge_time, seed_time):
    pad = 11776
    spread = jnp.arange(pad, dtype=jnp.int32) % N_NODES
    src = jnp.concatenate([edge_index[0], spread])
    dst = jnp.concatenate([edge_index[1], spread])
    et = jnp.concatenate([edge_time, jnp.full((pad,), 2 ** 30, jnp.int32)])
    pr, cr = _sc_call(x, src, dst, et, seed_time)
    p0 = pr[:N_NODES]
    p1 = pr[_NROWS:_NROWS + N_NODES]
    c0 = cr[:N_NODES].reshape(N_NODES // _R, 1, _R)
    c1 = cr[_NROWS:_NROWS + N_NODES].reshape(N_NODES // _R, 1, _R)
    return _combine(x, p0, p1, c0, c1)


# speedup vs baseline: 2.5110x; 1.5164x over previous
import functools

import jax
import jax.numpy as jnp
from jax import lax
from jax.experimental import pallas as pl
from jax.experimental.pallas import tpu as pltpu
from jax.experimental.pallas import tpu_sc as plsc

N_NODES = 10000
N_EDGES = 320000
D_FEAT = 128
TIME_WINDOW = 500

_B = 128
_K = 2
_TILES = 32
_NSLOT = 40
_EROWS = _NSLOT * _TILES * _K
_NROWS = 10240
_ZROWS = _NROWS // 16


def _sc_body(x_hbm, src_hbm, dst_hbm, et_hbm, st_hbm, p_out, c_out,
             acc, accc, srcv, dstv, etv, stv, deff, rows, onesv, zb2, zb1,
             s_idx, s_st, s_rows):
    cid = lax.axis_index("c")
    sid = lax.axis_index("s")
    wid = sid * 2 + cid

    z16 = jnp.zeros((16,), jnp.float32)
    for i in range(16):
        for j in range(8):
            zb2[i, pl.ds(j * 16, 16)] = z16
    for k in range(_ZROWS // 16):
        zb1[pl.ds(k * 16, 16)] = z16
    for j in range(8):
        onesv[pl.ds(j * 16, 16)] = jnp.ones((16,), jnp.float32)

    def zloop(k, carry):
        pltpu.sync_copy(zb2, acc.at[pl.ds(sid * _ZROWS + k * 16, 16)])
        return carry

    lax.fori_loop(0, _ZROWS // 16, zloop, None)
    pltpu.sync_copy(zb1, accc.at[pl.ds(sid * _ZROWS, _ZROWS)])

    plsc.subcore_barrier()

    dummy = N_NODES + sid * 15 + lax.rem(lax.iota(jnp.int32, 16),
                                         jnp.full((16,), 15, jnp.int32))

    def slot(g, carry):
        row0 = (g * _TILES + wid) * _K
        pltpu.make_async_copy(src_hbm.at[pl.ds(row0, _K)], srcv,
                              s_idx).start()
        pltpu.make_async_copy(dst_hbm.at[pl.ds(row0, _K)], dstv,
                              s_idx).start()
        pltpu.make_async_copy(et_hbm.at[pl.ds(row0, _K)], etv, s_idx).start()
        pltpu.make_async_copy(src_hbm.at[pl.ds(row0, _K)], srcv, s_idx).wait()
        pltpu.make_async_copy(dst_hbm.at[pl.ds(row0, _K)], dstv, s_idx).wait()
        pltpu.make_async_copy(et_hbm.at[pl.ds(row0, _K)], etv, s_idx).wait()
        for k in range(_K):
            pltpu.make_async_copy(st_hbm.at[dstv.at[k]], stv.at[k],
                                  s_st).start()
        for k in range(_K):
            pltpu.make_async_copy(st_hbm.at[dstv.at[k]], stv.at[k],
                                  s_st).wait()
        for k in range(_K):
            pltpu.make_async_copy(x_hbm.at[srcv.at[k]],
                                  rows.at[pl.ds(k * _B, _B)], s_rows).start()
        for k in range(_K):
            for j in range(_B // 16):
                sl = pl.ds(j * 16, 16)
                et = etv[k, sl]
                st = stv[k, sl]
                m = (et <= st) & (et > st - TIME_WINDOW)
                deff[k, sl] = jnp.where(m, dstv[k, sl], dummy)
        for k in range(_K):
            pltpu.make_async_copy(x_hbm.at[srcv.at[k]],
                                  rows.at[pl.ds(k * _B, _B)], s_rows).wait()
        for k in range(_K):
            pltpu.sync_copy(rows.at[pl.ds(k * _B, _B)], acc.at[deff.at[k]],
                            add=True)
            pltpu.sync_copy(onesv, accc.at[deff.at[k]], add=True)
        return carry

    lax.fori_loop(0, _NSLOT, slot, None)

    plsc.subcore_barrier()

    pltpu.sync_copy(acc.at[pl.ds(sid * _ZROWS, _ZROWS)],
                    p_out.at[pl.ds(cid * _NROWS + sid * _ZROWS, _ZROWS)])
    pltpu.sync_copy(accc.at[pl.ds(sid * _ZROWS, _ZROWS)],
                    c_out.at[pl.ds(cid * _NROWS + sid * _ZROWS, _ZROWS)])


_sc_call = functools.partial(
    pl.kernel,
    out_type=[
        jax.ShapeDtypeStruct((2 * _NROWS, D_FEAT), jnp.float32),
        jax.ShapeDtypeStruct((2 * _NROWS,), jnp.float32),
    ],
    mesh=plsc.VectorSubcoreMesh(core_axis_name="c", subcore_axis_name="s"),
    scratch_types=[
        pltpu.VMEM_SHARED((_NROWS, D_FEAT), jnp.float32),
        pltpu.VMEM_SHARED((_NROWS,), jnp.float32),
        pltpu.VMEM((_K, _B), jnp.int32),
        pltpu.VMEM((_K, _B), jnp.int32),
        pltpu.VMEM((_K, _B), jnp.int32),
        pltpu.VMEM((_K, _B), jnp.int32),
        pltpu.VMEM((_K, _B), jnp.int32),
        pltpu.VMEM((_K * _B, D_FEAT), jnp.float32),
        pltpu.VMEM((_B,), jnp.float32),
        pltpu.VMEM((16, D_FEAT), jnp.float32),
        pltpu.VMEM((_ZROWS,), jnp.float32),
        pltpu.SemaphoreType.DMA,
        pltpu.SemaphoreType.DMA,
        pltpu.SemaphoreType.DMA,
    ],
)(_sc_body)


def _combine_body(x_ref, p0_ref, p1_ref, c0_ref, c1_ref, o_ref):
    cnt = c0_ref[0, 0, :] + c1_ref[0, 0, :]
    s = p0_ref[...] + p1_ref[...]
    o_ref[...] = x_ref[...] + s / jnp.clip(cnt, 1.0, None)[:, None]


_R = 1000


def _combine(x, p0, p1, c0, c1):
    return pl.pallas_call(
        _combine_body,
        grid=(N_NODES // _R,),
        in_specs=[
            pl.BlockSpec((_R, D_FEAT), lambda i: (i, 0)),
            pl.BlockSpec((_R, D_FEAT), lambda i: (i, 0)),
            pl.BlockSpec((_R, D_FEAT), lambda i: (i, 0)),
            pl.BlockSpec((1, 1, _R), lambda i: (i, 0, 0)),
            pl.BlockSpec((1, 1, _R), lambda i: (i, 0, 0)),
        ],
        out_specs=pl.BlockSpec((_R, D_FEAT), lambda i: (i, 0)),
        out_shape=jax.ShapeDtypeStruct((N_NODES, D_FEAT), jnp.float32),
    )(x, p0, p1, c0, c1)


@jax.jit
def kernel(x, edge_index, edge_time, seed_time):
    pad = _EROWS * _B - N_EDGES
    spread = jnp.arange(pad, dtype=jnp.int32) % N_NODES
    src = jnp.concatenate([edge_index[0], spread]).reshape(_EROWS, _B)
    dst = jnp.concatenate([edge_index[1], spread]).reshape(_EROWS, _B)
    et = jnp.concatenate(
        [edge_time, jnp.full((pad,), 2 ** 30, jnp.int32)]).reshape(_EROWS, _B)
    pr, cr = _sc_call(x, src, dst, et, seed_time)
    p0 = pr[:N_NODES]
    p1 = pr[_NROWS:_NROWS + N_NODES]
    c0 = cr[:N_NODES].reshape(N_NODES // _R, 1, _R)
    c1 = cr[_NROWS:_NROWS + N_NODES].reshape(N_NODES // _R, 1, _R)
    return _combine(x, p0, p1, c0, c1)


# speedup vs baseline: 2.7511x; 1.0956x over previous
import functools

import jax
import jax.numpy as jnp
from jax import lax
from jax.experimental import pallas as pl
from jax.experimental.pallas import tpu as pltpu
from jax.experimental.pallas import tpu_sc as plsc

N_NODES = 10000
N_EDGES = 320000
D_FEAT = 128
TIME_WINDOW = 500

_B = 128
_K = 2
_TILES = 32
_NSLOT = 40
_EROWS = _NSLOT * _TILES * _K
_NROWS = 10240
_ZROWS = _NROWS // 16


def _sc_body(x_hbm, src_hbm, dst_hbm, et_hbm, st_hbm, p_out, c_out,
             acc, accc, srcv, dstv, etv, stv, deff, rows, onesv, zb2, zb1,
             s_idx, s_st, s_rows):
    cid = lax.axis_index("c")
    sid = lax.axis_index("s")
    wid = sid * 2 + cid

    z16 = jnp.zeros((16,), jnp.float32)
    for i in range(16):
        for j in range(8):
            zb2[i, pl.ds(j * 16, 16)] = z16
    for k in range(_ZROWS // 16):
        zb1[pl.ds(k * 16, 16)] = z16
    for j in range(8):
        onesv[pl.ds(j * 16, 16)] = jnp.ones((16,), jnp.float32)

    def zloop(k, carry):
        pltpu.sync_copy(zb2, acc.at[pl.ds(sid * _ZROWS + k * 16, 16)])
        return carry

    lax.fori_loop(0, _ZROWS // 16, zloop, None)
    pltpu.sync_copy(zb1, accc.at[pl.ds(sid * _ZROWS, _ZROWS)])

    plsc.subcore_barrier()

    dummy = N_NODES + sid * 15 + lax.rem(lax.iota(jnp.int32, 16),
                                         jnp.full((16,), 15, jnp.int32))

    def slot(g, carry):
        row0 = (g * _TILES + wid) * _K
        pltpu.make_async_copy(src_hbm.at[pl.ds(row0, _K)], srcv,
                              s_idx).start()
        pltpu.make_async_copy(dst_hbm.at[pl.ds(row0, _K)], dstv,
                              s_idx).start()
        pltpu.make_async_copy(et_hbm.at[pl.ds(row0, _K)], etv, s_idx).start()
        pltpu.make_async_copy(src_hbm.at[pl.ds(row0, _K)], srcv, s_idx).wait()
        pltpu.make_async_copy(dst_hbm.at[pl.ds(row0, _K)], dstv, s_idx).wait()
        pltpu.make_async_copy(et_hbm.at[pl.ds(row0, _K)], etv, s_idx).wait()
        for k in range(_K):
            pltpu.make_async_copy(st_hbm.at[dstv.at[k]], stv.at[k],
                                  s_st).start()
        for k in range(_K):
            pltpu.make_async_copy(st_hbm.at[dstv.at[k]], stv.at[k],
                                  s_st).wait()
        for k in range(_K):
            pltpu.make_async_copy(x_hbm.at[srcv.at[k]],
                                  rows.at[pl.ds(k * _B, _B)], s_rows).start()
        for k in range(_K):
            for j in range(_B // 16):
                sl = pl.ds(j * 16, 16)
                et = etv[k, sl]
                st = stv[k, sl]
                m = (et <= st) & (et > st - TIME_WINDOW)
                deff[k, sl] = jnp.where(m, dstv[k, sl], dummy)
        for k in range(_K):
            pltpu.make_async_copy(x_hbm.at[srcv.at[k]],
                                  rows.at[pl.ds(k * _B, _B)], s_rows).wait()
            pltpu.sync_copy(rows.at[pl.ds(k * _B, _B)], acc.at[deff.at[k]],
                            add=True)
            pltpu.sync_copy(onesv, accc.at[deff.at[k]], add=True)
        return carry

    lax.fori_loop(0, _NSLOT, slot, None)

    plsc.subcore_barrier()

    pltpu.sync_copy(acc.at[pl.ds(sid * _ZROWS, _ZROWS)],
                    p_out.at[pl.ds(cid * _NROWS + sid * _ZROWS, _ZROWS)])
    pltpu.sync_copy(accc.at[pl.ds(sid * _ZROWS, _ZROWS)],
                    c_out.at[pl.ds(cid * _NROWS + sid * _ZROWS, _ZROWS)])


_sc_call = functools.partial(
    pl.kernel,
    out_type=[
        jax.ShapeDtypeStruct((2 * _NROWS, D_FEAT), jnp.float32),
        jax.ShapeDtypeStruct((2 * _NROWS,), jnp.float32),
    ],
    mesh=plsc.VectorSubcoreMesh(core_axis_name="c", subcore_axis_name="s"),
    scratch_types=[
        pltpu.VMEM_SHARED((_NROWS, D_FEAT), jnp.float32),
        pltpu.VMEM_SHARED((_NROWS,), jnp.float32),
        pltpu.VMEM((_K, _B), jnp.int32),
        pltpu.VMEM((_K, _B), jnp.int32),
        pltpu.VMEM((_K, _B), jnp.int32),
        pltpu.VMEM((_K, _B), jnp.int32),
        pltpu.VMEM((_K, _B), jnp.int32),
        pltpu.VMEM((_K * _B, D_FEAT), jnp.float32),
        pltpu.VMEM((_B,), jnp.float32),
        pltpu.VMEM((16, D_FEAT), jnp.float32),
        pltpu.VMEM((_ZROWS,), jnp.float32),
        pltpu.SemaphoreType.DMA,
        pltpu.SemaphoreType.DMA,
        pltpu.SemaphoreType.DMA,
    ],
)(_sc_body)


def _combine_body(x_ref, p0_ref, p1_ref, c0_ref, c1_ref, o_ref):
    cnt = c0_ref[0, 0, :] + c1_ref[0, 0, :]
    s = p0_ref[...] + p1_ref[...]
    o_ref[...] = x_ref[...] + s / jnp.clip(cnt, 1.0, None)[:, None]


_R = 1000


def _combine(x, p0, p1, c0, c1):
    return pl.pallas_call(
        _combine_body,
        grid=(N_NODES // _R,),
        in_specs=[
            pl.BlockSpec((_R, D_FEAT), lambda i: (i, 0)),
            pl.BlockSpec((_R, D_FEAT), lambda i: (i, 0)),
            pl.BlockSpec((_R, D_FEAT), lambda i: (i, 0)),
            pl.BlockSpec((1, 1, _R), lambda i: (i, 0, 0)),
            pl.BlockSpec((1, 1, _R), lambda i: (i, 0, 0)),
        ],
        out_specs=pl.BlockSpec((_R, D_FEAT), lambda i: (i, 0)),
        out_shape=jax.ShapeDtypeStruct((N_NODES, D_FEAT), jnp.float32),
    )(x, p0, p1, c0, c1)


@jax.jit
def kernel(x, edge_index, edge_time, seed_time):
    pad = _EROWS * _B - N_EDGES
    spread = jnp.arange(pad, dtype=jnp.int32) % N_NODES
    src = jnp.concatenate([edge_index[0], spread]).reshape(_EROWS, _B)
    dst = jnp.concatenate([edge_index[1], spread]).reshape(_EROWS, _B)
    et = jnp.concatenate(
        [edge_time, jnp.full((pad,), 2 ** 30, jnp.int32)]).reshape(_EROWS, _B)
    pr, cr = _sc_call(x, src, dst, et, seed_time)
    p0 = pr[:N_NODES]
    p1 = pr[_NROWS:_NROWS + N_NODES]
    c0 = cr[:N_NODES].reshape(N_NODES // _R, 1, _R)
    c1 = cr[_NROWS:_NROWS + N_NODES].reshape(N_NODES // _R, 1, _R)
    return _combine(x, p0, p1, c0, c1)


# speedup vs baseline: 3.4906x; 1.2688x over previous
import functools

import jax
import jax.numpy as jnp
from jax import lax
from jax.experimental import pallas as pl
from jax.experimental.pallas import tpu as pltpu
from jax.experimental.pallas import tpu_sc as plsc

N_NODES = 10000
N_EDGES = 320000
D_FEAT = 128
TIME_WINDOW = 500

_B = 128
_K = 2
_TILES = 32
_NSLOT = 40
_EROWS = (_NSLOT + 1) * _TILES * _K
_NROWS = 10240
_ZROWS = _NROWS // 16


def _sc_body(x_hbm, src_hbm, dst_hbm, et_hbm, st_hbm, p_out, c_out,
             acc, accc, srcv, dstv, etv, stv, srcv1, dstv1, etv1, stv1,
             deff, rows, onesv, zb2, zb1,
             s_idx, s_st, s_idx1, s_st1, s_rows):
    cid = lax.axis_index("c")
    sid = lax.axis_index("s")
    wid = sid * 2 + cid

    z16 = jnp.zeros((16,), jnp.float32)
    for i in range(16):
        for j in range(8):
            zb2[i, pl.ds(j * 16, 16)] = z16
    for k in range(_ZROWS // 16):
        zb1[pl.ds(k * 16, 16)] = z16
    for j in range(8):
        onesv[pl.ds(j * 16, 16)] = jnp.ones((16,), jnp.float32)

    def zloop(k, carry):
        pltpu.sync_copy(zb2, acc.at[pl.ds(sid * _ZROWS + k * 16, 16)])
        return carry

    lax.fori_loop(0, _ZROWS // 16, zloop, None)
    pltpu.sync_copy(zb1, accc.at[pl.ds(sid * _ZROWS, _ZROWS)])

    plsc.subcore_barrier()

    dummy = N_NODES + sid * 15 + lax.rem(lax.iota(jnp.int32, 16),
                                         jnp.full((16,), 15, jnp.int32))

    bufs = [(srcv, dstv, etv, stv, s_idx, s_st),
            (srcv1, dstv1, etv1, stv1, s_idx1, s_st1)]

    def fire_idx(g, b):
        sv, dv, ev, _, si, _ = bufs[b]
        row0 = (g * _TILES + wid) * _K
        pltpu.make_async_copy(src_hbm.at[pl.ds(row0, _K)], sv, si).start()
        pltpu.make_async_copy(dst_hbm.at[pl.ds(row0, _K)], dv, si).start()
        pltpu.make_async_copy(et_hbm.at[pl.ds(row0, _K)], ev, si).start()

    def wait_idx(g, b):
        sv, dv, ev, _, si, _ = bufs[b]
        row0 = (g * _TILES + wid) * _K
        pltpu.make_async_copy(src_hbm.at[pl.ds(row0, _K)], sv, si).wait()
        pltpu.make_async_copy(dst_hbm.at[pl.ds(row0, _K)], dv, si).wait()
        pltpu.make_async_copy(et_hbm.at[pl.ds(row0, _K)], ev, si).wait()

    def fire_st(b):
        _, dv, _, tv, _, ss = bufs[b]
        for k in range(_K):
            pltpu.make_async_copy(st_hbm.at[dv.at[k]], tv.at[k], ss).start()

    def wait_st(b):
        _, dv, _, tv, _, ss = bufs[b]
        for k in range(_K):
            pltpu.make_async_copy(st_hbm.at[dv.at[k]], tv.at[k], ss).wait()

    def do_slot(g, b):
        sv, dv, ev, tv, _, _ = bufs[b]
        fire_idx(g + 1, b ^ 1)
        wait_st(b)
        for k in range(_K):
            pltpu.make_async_copy(x_hbm.at[sv.at[k]],
                                  rows.at[pl.ds(k * _B, _B)], s_rows).start()
        for k in range(_K):
            for j in range(_B // 16):
                sl = pl.ds(j * 16, 16)
                et = ev[k, sl]
                st = tv[k, sl]
                m = (et <= st) & (et > st - TIME_WINDOW)
                deff[k, sl] = jnp.where(m, dv[k, sl], dummy)
        wait_idx(g + 1, b ^ 1)
        fire_st(b ^ 1)
        for k in range(_K):
            pltpu.make_async_copy(x_hbm.at[sv.at[k]],
                                  rows.at[pl.ds(k * _B, _B)], s_rows).wait()
            pltpu.sync_copy(rows.at[pl.ds(k * _B, _B)], acc.at[deff.at[k]],
                            add=True)
            pltpu.sync_copy(onesv, accc.at[deff.at[k]], add=True)

    def pair(p, carry):
        do_slot(2 * p, 0)
        do_slot(2 * p + 1, 1)
        return carry

    fire_idx(0, 0)
    wait_idx(0, 0)
    fire_st(0)
    lax.fori_loop(0, _NSLOT // 2, pair, None)
    wait_st(0)

    plsc.subcore_barrier()

    pltpu.sync_copy(acc.at[pl.ds(sid * _ZROWS, _ZROWS)],
                    p_out.at[pl.ds(cid * _NROWS + sid * _ZROWS, _ZROWS)])
    pltpu.sync_copy(accc.at[pl.ds(sid * _ZROWS, _ZROWS)],
                    c_out.at[pl.ds(cid * _NROWS + sid * _ZROWS, _ZROWS)])


_sc_call = functools.partial(
    pl.kernel,
    out_type=[
        jax.ShapeDtypeStruct((2 * _NROWS, D_FEAT), jnp.float32),
        jax.ShapeDtypeStruct((2 * _NROWS,), jnp.float32),
    ],
    mesh=plsc.VectorSubcoreMesh(core_axis_name="c", subcore_axis_name="s"),
    scratch_types=[
        pltpu.VMEM_SHARED((_NROWS, D_FEAT), jnp.float32),
        pltpu.VMEM_SHARED((_NROWS,), jnp.float32),
        pltpu.VMEM((_K, _B), jnp.int32),
        pltpu.VMEM((_K, _B), jnp.int32),
        pltpu.VMEM((_K, _B), jnp.int32),
        pltpu.VMEM((_K, _B), jnp.int32),
        pltpu.VMEM((_K, _B), jnp.int32),
        pltpu.VMEM((_K, _B), jnp.int32),
        pltpu.VMEM((_K, _B), jnp.int32),
        pltpu.VMEM((_K, _B), jnp.int32),
        pltpu.VMEM((_K, _B), jnp.int32),
        pltpu.VMEM((_K * _B, D_FEAT), jnp.float32),
        pltpu.VMEM((_B,), jnp.float32),
        pltpu.VMEM((16, D_FEAT), jnp.float32),
        pltpu.VMEM((_ZROWS,), jnp.float32),
        pltpu.SemaphoreType.DMA,
        pltpu.SemaphoreType.DMA,
        pltpu.SemaphoreType.DMA,
        pltpu.SemaphoreType.DMA,
        pltpu.SemaphoreType.DMA,
    ],
)(_sc_body)


def _combine_body(x_ref, p0_ref, p1_ref, c0_ref, c1_ref, o_ref):
    cnt = c0_ref[0, 0, :] + c1_ref[0, 0, :]
    s = p0_ref[...] + p1_ref[...]
    o_ref[...] = x_ref[...] + s / jnp.clip(cnt, 1.0, None)[:, None]


_R = 1000


def _combine(x, p0, p1, c0, c1):
    return pl.pallas_call(
        _combine_body,
        grid=(N_NODES // _R,),
        in_specs=[
            pl.BlockSpec((_R, D_FEAT), lambda i: (i, 0)),
            pl.BlockSpec((_R, D_FEAT), lambda i: (i, 0)),
            pl.BlockSpec((_R, D_FEAT), lambda i: (i, 0)),
            pl.BlockSpec((1, 1, _R), lambda i: (i, 0, 0)),
            pl.BlockSpec((1, 1, _R), lambda i: (i, 0, 0)),
        ],
        out_specs=pl.BlockSpec((_R, D_FEAT), lambda i: (i, 0)),
        out_shape=jax.ShapeDtypeStruct((N_NODES, D_FEAT), jnp.float32),
    )(x, p0, p1, c0, c1)


@jax.jit
def kernel(x, edge_index, edge_time, seed_time):
    pad = _EROWS * _B - N_EDGES
    spread = jnp.arange(pad, dtype=jnp.int32) % N_NODES
    src = jnp.concatenate([edge_index[0], spread]).reshape(_EROWS, _B)
    dst = jnp.concatenate([edge_index[1], spread]).reshape(_EROWS, _B)
    et = jnp.concatenate(
        [edge_time, jnp.full((pad,), 2 ** 30, jnp.int32)]).reshape(_EROWS, _B)
    pr, cr = _sc_call(x, src, dst, et, seed_time)
    p0 = pr[:N_NODES]
    p1 = pr[_NROWS:_NROWS + N_NODES]
    c0 = cr[:N_NODES].reshape(N_NODES // _R, 1, _R)
    c1 = cr[_NROWS:_NROWS + N_NODES].reshape(N_NODES // _R, 1, _R)
    return _combine(x, p0, p1, c0, c1)


# speedup vs baseline: 4.0159x; 1.1505x over previous
import functools

import jax
import jax.numpy as jnp
from jax import lax
from jax.experimental import pallas as pl
from jax.experimental.pallas import tpu as pltpu
from jax.experimental.pallas import tpu_sc as plsc

N_NODES = 10000
N_EDGES = 320000
D_FEAT = 128
TIME_WINDOW = 500

_B = 128
_K = 2
_TILES = 32
_NSLOT = 40
_EROWS = (_NSLOT + 1) * _TILES * _K
_NROWS = 10240
_ZROWS = _NROWS // 16


def _sc_body(x_hbm, src_hbm, dst_hbm, et_hbm, st_hbm, p_out, c_out,
             acc, accc, srcv, dstv, etv, stv, srcv1, dstv1, etv1, stv1,
             deff, rows, onesv, zb2, zb1,
             s_idx, s_st, s_idx1, s_st1, s_rows):
    cid = lax.axis_index("c")
    sid = lax.axis_index("s")
    wid = sid * 2 + cid

    z16 = jnp.zeros((16,), jnp.float32)
    for i in range(16):
        for j in range(8):
            zb2[i, pl.ds(j * 16, 16)] = z16
    for k in range(_ZROWS // 16):
        zb1[pl.ds(k * 16, 16)] = z16
    for j in range(8):
        onesv[pl.ds(j * 16, 16)] = jnp.ones((16,), jnp.float32)

    def zloop(k, carry):
        pltpu.sync_copy(zb2, acc.at[pl.ds(sid * _ZROWS + k * 16, 16)])
        return carry

    lax.fori_loop(0, _ZROWS // 16, zloop, None)
    pltpu.sync_copy(zb1, accc.at[pl.ds(sid * _ZROWS, _ZROWS)])

    plsc.subcore_barrier()

    dummy = N_NODES + sid * 15 + lax.rem(lax.iota(jnp.int32, 16),
                                         jnp.full((16,), 15, jnp.int32))

    bufs = [(srcv, dstv, etv, stv, s_idx, s_st),
            (srcv1, dstv1, etv1, stv1, s_idx1, s_st1)]

    def fire_idx(g, b):
        sv, dv, ev, _, si, _ = bufs[b]
        row0 = (g * _TILES + wid) * _K
        pltpu.make_async_copy(src_hbm.at[pl.ds(row0, _K)], sv, si).start()
        pltpu.make_async_copy(dst_hbm.at[pl.ds(row0, _K)], dv, si).start()
        pltpu.make_async_copy(et_hbm.at[pl.ds(row0, _K)], ev, si).start()

    def wait_idx(g, b):
        sv, dv, ev, _, si, _ = bufs[b]
        row0 = (g * _TILES + wid) * _K
        pltpu.make_async_copy(src_hbm.at[pl.ds(row0, _K)], sv, si).wait()
        pltpu.make_async_copy(dst_hbm.at[pl.ds(row0, _K)], dv, si).wait()
        pltpu.make_async_copy(et_hbm.at[pl.ds(row0, _K)], ev, si).wait()

    def fire_st(b):
        _, dv, _, tv, _, ss = bufs[b]
        for k in range(_K):
            pltpu.make_async_copy(st_hbm.at[dv.at[k]], tv.at[k], ss).start()

    def wait_st(b):
        _, dv, _, tv, _, ss = bufs[b]
        for k in range(_K):
            pltpu.make_async_copy(st_hbm.at[dv.at[k]], tv.at[k], ss).wait()

    def fire_rows(b, k):
        sv = bufs[b][0]
        pltpu.make_async_copy(x_hbm.at[sv.at[k]],
                              rows.at[pl.ds(k * _B, _B)], s_rows).start()

    def wait_rows(b, k):
        sv = bufs[b][0]
        pltpu.make_async_copy(x_hbm.at[sv.at[k]],
                              rows.at[pl.ds(k * _B, _B)], s_rows).wait()

    def do_slot(g, b):
        dv, ev, tv = bufs[b][1], bufs[b][2], bufs[b][3]
        fire_idx(g + 1, b ^ 1)
        wait_st(b)
        for k in range(_K):
            for j in range(_B // 16):
                sl = pl.ds(j * 16, 16)
                et = ev[k, sl]
                st = tv[k, sl]
                m = (et <= st) & (et > st - TIME_WINDOW)
                deff[k, sl] = jnp.where(m, dv[k, sl], dummy)
        wait_idx(g + 1, b ^ 1)
        fire_st(b ^ 1)
        for k in range(_K):
            wait_rows(b, k)
            pltpu.sync_copy(rows.at[pl.ds(k * _B, _B)], acc.at[deff.at[k]],
                            add=True)
            pltpu.sync_copy(onesv, accc.at[deff.at[k]], add=True)
            fire_rows(b ^ 1, k)

    def pair(p, carry):
        do_slot(2 * p, 0)
        do_slot(2 * p + 1, 1)
        return carry

    fire_idx(0, 0)
    wait_idx(0, 0)
    fire_st(0)
    for k in range(_K):
        fire_rows(0, k)
    lax.fori_loop(0, _NSLOT // 2, pair, None)
    wait_st(0)
    for k in range(_K):
        wait_rows(0, k)

    plsc.subcore_barrier()

    pltpu.sync_copy(acc.at[pl.ds(sid * _ZROWS, _ZROWS)],
                    p_out.at[pl.ds(cid * _NROWS + sid * _ZROWS, _ZROWS)])
    pltpu.sync_copy(accc.at[pl.ds(sid * _ZROWS, _ZROWS)],
                    c_out.at[pl.ds(cid * _NROWS + sid * _ZROWS, _ZROWS)])


_sc_call = functools.partial(
    pl.kernel,
    out_type=[
        jax.ShapeDtypeStruct((2 * _NROWS, D_FEAT), jnp.float32),
        jax.ShapeDtypeStruct((2 * _NROWS,), jnp.float32),
    ],
    mesh=plsc.VectorSubcoreMesh(core_axis_name="c", subcore_axis_name="s"),
    scratch_types=[
        pltpu.VMEM_SHARED((_NROWS, D_FEAT), jnp.float32),
        pltpu.VMEM_SHARED((_NROWS,), jnp.float32),
        pltpu.VMEM((_K, _B), jnp.int32),
        pltpu.VMEM((_K, _B), jnp.int32),
        pltpu.VMEM((_K, _B), jnp.int32),
        pltpu.VMEM((_K, _B), jnp.int32),
        pltpu.VMEM((_K, _B), jnp.int32),
        pltpu.VMEM((_K, _B), jnp.int32),
        pltpu.VMEM((_K, _B), jnp.int32),
        pltpu.VMEM((_K, _B), jnp.int32),
        pltpu.VMEM((_K, _B), jnp.int32),
        pltpu.VMEM((_K * _B, D_FEAT), jnp.float32),
        pltpu.VMEM((_B,), jnp.float32),
        pltpu.VMEM((16, D_FEAT), jnp.float32),
        pltpu.VMEM((_ZROWS,), jnp.float32),
        pltpu.SemaphoreType.DMA,
        pltpu.SemaphoreType.DMA,
        pltpu.SemaphoreType.DMA,
        pltpu.SemaphoreType.DMA,
        pltpu.SemaphoreType.DMA,
    ],
)(_sc_body)


def _combine_body(x_ref, p0_ref, p1_ref, c0_ref, c1_ref, o_ref):
    cnt = c0_ref[0, 0, :] + c1_ref[0, 0, :]
    s = p0_ref[...] + p1_ref[...]
    o_ref[...] = x_ref[...] + s / jnp.clip(cnt, 1.0, None)[:, None]


_R = 1000


def _combine(x, p0, p1, c0, c1):
    return pl.pallas_call(
        _combine_body,
        grid=(N_NODES // _R,),
        in_specs=[
            pl.BlockSpec((_R, D_FEAT), lambda i: (i, 0)),
            pl.BlockSpec((_R, D_FEAT), lambda i: (i, 0)),
            pl.BlockSpec((_R, D_FEAT), lambda i: (i, 0)),
            pl.BlockSpec((1, 1, _R), lambda i: (i, 0, 0)),
            pl.BlockSpec((1, 1, _R), lambda i: (i, 0, 0)),
        ],
        out_specs=pl.BlockSpec((_R, D_FEAT), lambda i: (i, 0)),
        out_shape=jax.ShapeDtypeStruct((N_NODES, D_FEAT), jnp.float32),
    )(x, p0, p1, c0, c1)


@jax.jit
def kernel(x, edge_index, edge_time, seed_time):
    pad = _EROWS * _B - N_EDGES
    spread = jnp.arange(pad, dtype=jnp.int32) % N_NODES
    src = jnp.concatenate([edge_index[0], spread]).reshape(_EROWS, _B)
    dst = jnp.concatenate([edge_index[1], spread]).reshape(_EROWS, _B)
    et = jnp.concatenate(
        [edge_time, jnp.full((pad,), 2 ** 30, jnp.int32)]).reshape(_EROWS, _B)
    pr, cr = _sc_call(x, src, dst, et, seed_time)
    p0 = pr[:N_NODES]
    p1 = pr[_NROWS:_NROWS + N_NODES]
    c0 = cr[:N_NODES].reshape(N_NODES // _R, 1, _R)
    c1 = cr[_NROWS:_NROWS + N_NODES].reshape(N_NODES // _R, 1, _R)
    return _combine(x, p0, p1, c0, c1)


# speedup vs baseline: 4.0268x; 1.0027x over previous
import functools

import jax
import jax.numpy as jnp
from jax import lax
from jax.experimental import pallas as pl
from jax.experimental.pallas import tpu as pltpu
from jax.experimental.pallas import tpu_sc as plsc

N_NODES = 10000
N_EDGES = 320000
D_FEAT = 128
TIME_WINDOW = 500

_B = 128
_K = 2
_TILES = 32
_NSLOT = 40
_EROWS = (_NSLOT + 1) * _TILES * _K
_NROWS = 10240
_ZROWS = _NROWS // 16


def _sc_body(x_hbm, src_hbm, dst_hbm, et_hbm, st_hbm, p_out, c_out,
             acc, accc, srcv, dstv, etv, stv, srcv1, dstv1, etv1, stv1,
             deff, rows, onesv, zb2, zb1,
             s_idx, s_st, s_idx1, s_st1, s_rows):
    cid = lax.axis_index("c")
    sid = lax.axis_index("s")
    wid = sid * 2 + cid

    z16 = jnp.zeros((16,), jnp.float32)
    for i in range(16):
        for j in range(8):
            zb2[i, pl.ds(j * 16, 16)] = z16
    for k in range(_ZROWS // 16):
        zb1[pl.ds(k * 16, 16)] = z16
    for j in range(8):
        onesv[pl.ds(j * 16, 16)] = jnp.ones((16,), jnp.float32)

    def zloop(k, carry):
        pltpu.sync_copy(zb2, acc.at[pl.ds(sid * _ZROWS + k * 16, 16)])
        return carry

    lax.fori_loop(0, _ZROWS // 16, zloop, None)
    pltpu.sync_copy(zb1, accc.at[pl.ds(sid * _ZROWS, _ZROWS)])

    plsc.subcore_barrier()

    iota16 = lax.iota(jnp.int32, 16)
    c240 = jnp.full((16,), 240, jnp.int32)

    def dummy_rows(k, j):
        rot = iota16 + jnp.full((16,), 16 * (8 * k + j), jnp.int32) + sid
        return N_NODES + lax.rem(rot, c240)

    bufs = [(srcv, dstv, etv, stv, s_idx, s_st),
            (srcv1, dstv1, etv1, stv1, s_idx1, s_st1)]

    def fire_idx(g, b):
        sv, dv, ev, _, si, _ = bufs[b]
        row0 = (g * _TILES + wid) * _K
        pltpu.make_async_copy(src_hbm.at[pl.ds(row0, _K)], sv, si).start()
        pltpu.make_async_copy(dst_hbm.at[pl.ds(row0, _K)], dv, si).start()
        pltpu.make_async_copy(et_hbm.at[pl.ds(row0, _K)], ev, si).start()

    def wait_idx(g, b):
        sv, dv, ev, _, si, _ = bufs[b]
        row0 = (g * _TILES + wid) * _K
        pltpu.make_async_copy(src_hbm.at[pl.ds(row0, _K)], sv, si).wait()
        pltpu.make_async_copy(dst_hbm.at[pl.ds(row0, _K)], dv, si).wait()
        pltpu.make_async_copy(et_hbm.at[pl.ds(row0, _K)], ev, si).wait()

    def fire_st(b):
        _, dv, _, tv, _, ss = bufs[b]
        for k in range(_K):
            pltpu.make_async_copy(st_hbm.at[dv.at[k]], tv.at[k], ss).start()

    def wait_st(b):
        _, dv, _, tv, _, ss = bufs[b]
        for k in range(_K):
            pltpu.make_async_copy(st_hbm.at[dv.at[k]], tv.at[k], ss).wait()

    def fire_rows(b, k):
        sv = bufs[b][0]
        pltpu.make_async_copy(x_hbm.at[sv.at[k]],
                              rows.at[pl.ds(k * _B, _B)], s_rows).start()

    def wait_rows(b, k):
        sv = bufs[b][0]
        pltpu.make_async_copy(x_hbm.at[sv.at[k]],
                              rows.at[pl.ds(k * _B, _B)], s_rows).wait()

    def do_slot(g, b):
        dv, ev, tv = bufs[b][1], bufs[b][2], bufs[b][3]
        fire_idx(g + 1, b ^ 1)
        wait_st(b)
        for k in range(_K):
            for j in range(_B // 16):
                sl = pl.ds(j * 16, 16)
                et = ev[k, sl]
                st = tv[k, sl]
                m = (et <= st) & (et > st - TIME_WINDOW)
                deff[k, sl] = jnp.where(m, dv[k, sl], dummy_rows(k, j))
        wait_idx(g + 1, b ^ 1)
        fire_st(b ^ 1)
        for k in range(_K):
            wait_rows(b, k)
            pltpu.sync_copy(rows.at[pl.ds(k * _B, _B)], acc.at[deff.at[k]],
                            add=True)
            pltpu.sync_copy(onesv, accc.at[deff.at[k]], add=True)
            fire_rows(b ^ 1, k)

    def pair(p, carry):
        do_slot(2 * p, 0)
        do_slot(2 * p + 1, 1)
        return carry

    fire_idx(0, 0)
    wait_idx(0, 0)
    fire_st(0)
    for k in range(_K):
        fire_rows(0, k)
    lax.fori_loop(0, _NSLOT // 2, pair, None)
    wait_st(0)
    for k in range(_K):
        wait_rows(0, k)

    plsc.subcore_barrier()

    pltpu.sync_copy(acc.at[pl.ds(sid * _ZROWS, _ZROWS)],
                    p_out.at[pl.ds(cid * _NROWS + sid * _ZROWS, _ZROWS)])
    pltpu.sync_copy(accc.at[pl.ds(sid * _ZROWS, _ZROWS)],
                    c_out.at[pl.ds(cid * _NROWS + sid * _ZROWS, _ZROWS)])


_sc_call = functools.partial(
    pl.kernel,
    out_type=[
        jax.ShapeDtypeStruct((2 * _NROWS, D_FEAT), jnp.float32),
        jax.ShapeDtypeStruct((2 * _NROWS,), jnp.float32),
    ],
    mesh=plsc.VectorSubcoreMesh(core_axis_name="c", subcore_axis_name="s"),
    scratch_types=[
        pltpu.VMEM_SHARED((_NROWS, D_FEAT), jnp.float32),
        pltpu.VMEM_SHARED((_NROWS,), jnp.float32),
        pltpu.VMEM((_K, _B), jnp.int32),
        pltpu.VMEM((_K, _B), jnp.int32),
        pltpu.VMEM((_K, _B), jnp.int32),
        pltpu.VMEM((_K, _B), jnp.int32),
        pltpu.VMEM((_K, _B), jnp.int32),
        pltpu.VMEM((_K, _B), jnp.int32),
        pltpu.VMEM((_K, _B), jnp.int32),
        pltpu.VMEM((_K, _B), jnp.int32),
        pltpu.VMEM((_K, _B), jnp.int32),
        pltpu.VMEM((_K * _B, D_FEAT), jnp.float32),
        pltpu.VMEM((_B,), jnp.float32),
        pltpu.VMEM((16, D_FEAT), jnp.float32),
        pltpu.VMEM((_ZROWS,), jnp.float32),
        pltpu.SemaphoreType.DMA,
        pltpu.SemaphoreType.DMA,
        pltpu.SemaphoreType.DMA,
        pltpu.SemaphoreType.DMA,
        pltpu.SemaphoreType.DMA,
    ],
)(_sc_body)


def _combine_body(x_ref, p0_ref, p1_ref, c0_ref, c1_ref, o_ref):
    cnt = c0_ref[0, 0, :] + c1_ref[0, 0, :]
    s = p0_ref[...] + p1_ref[...]
    o_ref[...] = x_ref[...] + s / jnp.clip(cnt, 1.0, None)[:, None]


_R = 1000


def _combine(x, p0, p1, c0, c1):
    return pl.pallas_call(
        _combine_body,
        grid=(N_NODES // _R,),
        in_specs=[
            pl.BlockSpec((_R, D_FEAT), lambda i: (i, 0)),
            pl.BlockSpec((_R, D_FEAT), lambda i: (i, 0)),
            pl.BlockSpec((_R, D_FEAT), lambda i: (i, 0)),
            pl.BlockSpec((1, 1, _R), lambda i: (i, 0, 0)),
            pl.BlockSpec((1, 1, _R), lambda i: (i, 0, 0)),
        ],
        out_specs=pl.BlockSpec((_R, D_FEAT), lambda i: (i, 0)),
        out_shape=jax.ShapeDtypeStruct((N_NODES, D_FEAT), jnp.float32),
    )(x, p0, p1, c0, c1)


@jax.jit
def kernel(x, edge_index, edge_time, seed_time):
    pad = _EROWS * _B - N_EDGES
    spread = jnp.arange(pad, dtype=jnp.int32) % N_NODES
    src = jnp.concatenate([edge_index[0], spread]).reshape(_EROWS, _B)
    dst = jnp.concatenate([edge_index[1], spread]).reshape(_EROWS, _B)
    et = jnp.concatenate(
        [edge_time, jnp.full((pad,), 2 ** 30, jnp.int32)]).reshape(_EROWS, _B)
    pr, cr = _sc_call(x, src, dst, et, seed_time)
    p0 = pr[:N_NODES]
    p1 = pr[_NROWS:_NROWS + N_NODES]
    c0 = cr[:N_NODES].reshape(N_NODES // _R, 1, _R)
    c1 = cr[_NROWS:_NROWS + N_NODES].reshape(N_NODES // _R, 1, _R)
    return _combine(x, p0, p1, c0, c1)


# speedup vs baseline: 4.0460x; 1.0048x over previous
import functools

import jax
import jax.numpy as jnp
from jax import lax
from jax.experimental import pallas as pl
from jax.experimental.pallas import tpu as pltpu
from jax.experimental.pallas import tpu_sc as plsc

N_NODES = 10000
N_EDGES = 320000
D_FEAT = 128
TIME_WINDOW = 500

_B = 128
_K = 2
_TILES = 32
_NSLOT = 40
_EROWS = (_NSLOT + 1) * _TILES * _K
_NROWS = 10240
_ZROWS = _NROWS // 16


def _sc_body(x_hbm, src_hbm, dst_hbm, et_hbm, st_hbm, p_out, c_out,
             acc, accc, srcv, dstv, etv, stv, srcv1, dstv1, etv1, stv1,
             deff, deff1, rows, onesv, zb2, zb1,
             s_idx, s_st, s_cnt, s_idx1, s_st1, s_cnt1, s_rows):
    cid = lax.axis_index("c")
    sid = lax.axis_index("s")
    wid = sid * 2 + cid

    z16 = jnp.zeros((16,), jnp.float32)
    for i in range(16):
        for j in range(8):
            zb2[i, pl.ds(j * 16, 16)] = z16
    for k in range(_ZROWS // 16):
        zb1[pl.ds(k * 16, 16)] = z16
    for j in range(8):
        onesv[pl.ds(j * 16, 16)] = jnp.ones((16,), jnp.float32)

    def zloop(k, carry):
        pltpu.sync_copy(zb2, acc.at[pl.ds(sid * _ZROWS + k * 16, 16)])
        return carry

    lax.fori_loop(0, _ZROWS // 16, zloop, None)
    pltpu.sync_copy(zb1, accc.at[pl.ds(sid * _ZROWS, _ZROWS)])

    plsc.subcore_barrier()

    iota16 = lax.iota(jnp.int32, 16)
    c240 = jnp.full((16,), 240, jnp.int32)

    def dummy_rows(k, j):
        rot = iota16 + jnp.full((16,), 16 * (8 * k + j), jnp.int32) + sid
        return N_NODES + lax.rem(rot, c240)

    bufs = [(srcv, dstv, etv, stv, s_idx, s_st, deff, s_cnt),
            (srcv1, dstv1, etv1, stv1, s_idx1, s_st1, deff1, s_cnt1)]

    def fire_idx(g, b):
        sv, dv, ev, si = bufs[b][0], bufs[b][1], bufs[b][2], bufs[b][4]
        row0 = (g * _TILES + wid) * _K
        pltpu.make_async_copy(src_hbm.at[pl.ds(row0, _K)], sv, si).start()
        pltpu.make_async_copy(dst_hbm.at[pl.ds(row0, _K)], dv, si).start()
        pltpu.make_async_copy(et_hbm.at[pl.ds(row0, _K)], ev, si).start()

    def wait_idx(g, b):
        sv, dv, ev, si = bufs[b][0], bufs[b][1], bufs[b][2], bufs[b][4]
        row0 = (g * _TILES + wid) * _K
        pltpu.make_async_copy(src_hbm.at[pl.ds(row0, _K)], sv, si).wait()
        pltpu.make_async_copy(dst_hbm.at[pl.ds(row0, _K)], dv, si).wait()
        pltpu.make_async_copy(et_hbm.at[pl.ds(row0, _K)], ev, si).wait()

    def fire_st(b):
        dv, tv, ss = bufs[b][1], bufs[b][3], bufs[b][5]
        for k in range(_K):
            pltpu.make_async_copy(st_hbm.at[dv.at[k]], tv.at[k], ss).start()

    def wait_st(b):
        dv, tv, ss = bufs[b][1], bufs[b][3], bufs[b][5]
        for k in range(_K):
            pltpu.make_async_copy(st_hbm.at[dv.at[k]], tv.at[k], ss).wait()

    def fire_rows(b, k):
        sv = bufs[b][0]
        pltpu.make_async_copy(x_hbm.at[sv.at[k]],
                              rows.at[pl.ds(k * _B, _B)], s_rows).start()

    def wait_rows(b, k):
        sv = bufs[b][0]
        pltpu.make_async_copy(x_hbm.at[sv.at[k]],
                              rows.at[pl.ds(k * _B, _B)], s_rows).wait()

    def do_slot(g, b, first=False):
        dv, ev, tv, de, sc = (bufs[b][1], bufs[b][2], bufs[b][3], bufs[b][6],
                              bufs[b][7])
        fire_idx(g + 1, b ^ 1)
        wait_st(b)
        if not first:
            for k in range(_K):
                pltpu.make_async_copy(onesv, accc.at[de.at[k]], sc).wait()
        for k in range(_K):
            for j in range(_B // 16):
                sl = pl.ds(j * 16, 16)
                et = ev[k, sl]
                st = tv[k, sl]
                m = (et <= st) & (et > st - TIME_WINDOW)
                de[k, sl] = jnp.where(m, dv[k, sl], dummy_rows(k, j))
        wait_idx(g + 1, b ^ 1)
        fire_st(b ^ 1)
        for k in range(_K):
            wait_rows(b, k)
            pltpu.sync_copy(rows.at[pl.ds(k * _B, _B)], acc.at[de.at[k]],
                            add=True)
            pltpu.make_async_copy(onesv, accc.at[de.at[k]], sc).start()
            fire_rows(b ^ 1, k)

    def pair(p, carry):
        do_slot(2 * p, 0)
        do_slot(2 * p + 1, 1)
        return carry

    fire_idx(0, 0)
    wait_idx(0, 0)
    fire_st(0)
    for k in range(_K):
        fire_rows(0, k)
    do_slot(0, 0, first=True)
    do_slot(1, 1, first=True)
    lax.fori_loop(1, _NSLOT // 2, pair, None)
    wait_st(0)
    for k in range(_K):
        wait_rows(0, k)
    for b in (0, 1):
        de, sc = bufs[b][6], bufs[b][7]
        for k in range(_K):
            pltpu.make_async_copy(onesv, accc.at[de.at[k]], sc).wait()

    plsc.subcore_barrier()

    pltpu.sync_copy(acc.at[pl.ds(sid * _ZROWS, _ZROWS)],
                    p_out.at[pl.ds(cid * _NROWS + sid * _ZROWS, _ZROWS)])
    pltpu.sync_copy(accc.at[pl.ds(sid * _ZROWS, _ZROWS)],
                    c_out.at[pl.ds(cid * _NROWS + sid * _ZROWS, _ZROWS)])


_sc_call = functools.partial(
    pl.kernel,
    out_type=[
        jax.ShapeDtypeStruct((2 * _NROWS, D_FEAT), jnp.float32),
        jax.ShapeDtypeStruct((2 * _NROWS,), jnp.float32),
    ],
    mesh=plsc.VectorSubcoreMesh(core_axis_name="c", subcore_axis_name="s"),
    scratch_types=[
        pltpu.VMEM_SHARED((_NROWS, D_FEAT), jnp.float32),
        pltpu.VMEM_SHARED((_NROWS,), jnp.float32),
        pltpu.VMEM((_K, _B), jnp.int32),
        pltpu.VMEM((_K, _B), jnp.int32),
        pltpu.VMEM((_K, _B), jnp.int32),
        pltpu.VMEM((_K, _B), jnp.int32),
        pltpu.VMEM((_K, _B), jnp.int32),
        pltpu.VMEM((_K, _B), jnp.int32),
        pltpu.VMEM((_K, _B), jnp.int32),
        pltpu.VMEM((_K, _B), jnp.int32),
        pltpu.VMEM((_K, _B), jnp.int32),
        pltpu.VMEM((_K, _B), jnp.int32),
        pltpu.VMEM((_K * _B, D_FEAT), jnp.float32),
        pltpu.VMEM((_B,), jnp.float32),
        pltpu.VMEM((16, D_FEAT), jnp.float32),
        pltpu.VMEM((_ZROWS,), jnp.float32),
        pltpu.SemaphoreType.DMA,
        pltpu.SemaphoreType.DMA,
        pltpu.SemaphoreType.DMA,
        pltpu.SemaphoreType.DMA,
        pltpu.SemaphoreType.DMA,
        pltpu.SemaphoreType.DMA,
        pltpu.SemaphoreType.DMA,
    ],
)(_sc_body)


def _combine_body(x_ref, p0_ref, p1_ref, c0_ref, c1_ref, o_ref):
    cnt = c0_ref[0, 0, :] + c1_ref[0, 0, :]
    s = p0_ref[...] + p1_ref[...]
    o_ref[...] = x_ref[...] + s / jnp.clip(cnt, 1.0, None)[:, None]


_R = 1000


def _combine(x, p0, p1, c0, c1):
    return pl.pallas_call(
        _combine_body,
        grid=(N_NODES // _R,),
        in_specs=[
            pl.BlockSpec((_R, D_FEAT), lambda i: (i, 0)),
            pl.BlockSpec((_R, D_FEAT), lambda i: (i, 0)),
            pl.BlockSpec((_R, D_FEAT), lambda i: (i, 0)),
            pl.BlockSpec((1, 1, _R), lambda i: (i, 0, 0)),
            pl.BlockSpec((1, 1, _R), lambda i: (i, 0, 0)),
        ],
        out_specs=pl.BlockSpec((_R, D_FEAT), lambda i: (i, 0)),
        out_shape=jax.ShapeDtypeStruct((N_NODES, D_FEAT), jnp.float32),
    )(x, p0, p1, c0, c1)


@jax.jit
def kernel(x, edge_index, edge_time, seed_time):
    pad = _EROWS * _B - N_EDGES
    spread = jnp.arange(pad, dtype=jnp.int32) % N_NODES
    src = jnp.concatenate([edge_index[0], spread]).reshape(_EROWS, _B)
    dst = jnp.concatenate([edge_index[1], spread]).reshape(_EROWS, _B)
    et = jnp.concatenate(
        [edge_time, jnp.full((pad,), 2 ** 30, jnp.int32)]).reshape(_EROWS, _B)
    pr, cr = _sc_call(x, src, dst, et, seed_time)
    p0 = pr[:N_NODES]
    p1 = pr[_NROWS:_NROWS + N_NODES]
    c0 = cr[:N_NODES].reshape(N_NODES // _R, 1, _R)
    c1 = cr[_NROWS:_NROWS + N_NODES].reshape(N_NODES // _R, 1, _R)
    return _combine(x, p0, p1, c0, c1)


# speedup vs baseline: 4.0721x; 1.0064x over previous
import functools

import jax
import jax.numpy as jnp
from jax import lax
from jax.experimental import pallas as pl
from jax.experimental.pallas import tpu as pltpu
from jax.experimental.pallas import tpu_sc as plsc

N_NODES = 10000
N_EDGES = 320000
D_FEAT = 128
TIME_WINDOW = 500

_B = 128
_K = 2
_TILES = 32
_NSLOT = 40
_EROWS = (_NSLOT + 1) * _TILES * _K
_NROWS = 10240
_ZROWS = _NROWS // 16


def _sc_body(x_hbm, src_hbm, dst_hbm, et_hbm, st_hbm, p_out, c_out,
             acc, accc, srcv, dstv, etv, stv, srcv1, dstv1, etv1, stv1,
             deff, rows, onesv, zb2, zb1,
             s_idx, s_st, s_idx1, s_st1, s_rows, s_sc):
    cid = lax.axis_index("c")
    sid = lax.axis_index("s")
    wid = sid * 2 + cid

    z16 = jnp.zeros((16,), jnp.float32)
    for i in range(16):
        for j in range(8):
            zb2[i, pl.ds(j * 16, 16)] = z16
    for k in range(_ZROWS // 16):
        zb1[pl.ds(k * 16, 16)] = z16
    for j in range(8):
        onesv[pl.ds(j * 16, 16)] = jnp.ones((16,), jnp.float32)

    def zloop(k, carry):
        pltpu.sync_copy(zb2, acc.at[pl.ds(sid * _ZROWS + k * 16, 16)])
        return carry

    lax.fori_loop(0, _ZROWS // 16, zloop, None)
    pltpu.sync_copy(zb1, accc.at[pl.ds(sid * _ZROWS, _ZROWS)])

    plsc.subcore_barrier()

    iota16 = lax.iota(jnp.int32, 16)
    c240 = jnp.full((16,), 240, jnp.int32)

    def dummy_rows(k, j):
        rot = iota16 + jnp.full((16,), 16 * (8 * k + j), jnp.int32) + sid
        return N_NODES + lax.rem(rot, c240)

    bufs = [(srcv, dstv, etv, stv, s_idx, s_st),
            (srcv1, dstv1, etv1, stv1, s_idx1, s_st1)]

    def fire_idx(g, b):
        sv, dv, ev, _, si, _ = bufs[b]
        row0 = (g * _TILES + wid) * _K
        pltpu.make_async_copy(src_hbm.at[pl.ds(row0, _K)], sv, si).start()
        pltpu.make_async_copy(dst_hbm.at[pl.ds(row0, _K)], dv, si).start()
        pltpu.make_async_copy(et_hbm.at[pl.ds(row0, _K)], ev, si).start()

    def wait_idx(g, b):
        sv, dv, ev, _, si, _ = bufs[b]
        row0 = (g * _TILES + wid) * _K
        pltpu.make_async_copy(src_hbm.at[pl.ds(row0, _K)], sv, si).wait()
        pltpu.make_async_copy(dst_hbm.at[pl.ds(row0, _K)], dv, si).wait()
        pltpu.make_async_copy(et_hbm.at[pl.ds(row0, _K)], ev, si).wait()

    def fire_st(b):
        _, dv, _, tv, _, ss = bufs[b]
        for k in range(_K):
            pltpu.make_async_copy(st_hbm.at[dv.at[k]], tv.at[k], ss).start()

    def wait_st(b):
        _, dv, _, tv, _, ss = bufs[b]
        for k in range(_K):
            pltpu.make_async_copy(st_hbm.at[dv.at[k]], tv.at[k], ss).wait()

    def fire_rows(b, k):
        sv = bufs[b][0]
        pltpu.make_async_copy(x_hbm.at[sv.at[k]],
                              rows.at[pl.ds(k * _B, _B)], s_rows).start()

    def wait_rows(b, k):
        sv = bufs[b][0]
        pltpu.make_async_copy(x_hbm.at[sv.at[k]],
                              rows.at[pl.ds(k * _B, _B)], s_rows).wait()

    def do_slot(g, b):
        dv, ev, tv = bufs[b][1], bufs[b][2], bufs[b][3]
        fire_idx(g + 1, b ^ 1)
        wait_st(b)
        for k in range(_K):
            for j in range(_B // 16):
                sl = pl.ds(j * 16, 16)
                et = ev[k, sl]
                st = tv[k, sl]
                m = (et <= st) & (et > st - TIME_WINDOW)
                deff[k, sl] = jnp.where(m, dv[k, sl], dummy_rows(k, j))
        wait_idx(g + 1, b ^ 1)
        fire_st(b ^ 1)
        for k in range(_K):
            wait_rows(b, k)
            pltpu.make_async_copy(rows.at[pl.ds(k * _B, _B)],
                                  acc.at[deff.at[k]], s_sc).start()
            pltpu.make_async_copy(onesv, accc.at[deff.at[k]], s_sc).start()
        for k in range(_K):
            pltpu.make_async_copy(rows.at[pl.ds(k * _B, _B)],
                                  acc.at[deff.at[k]], s_sc).wait()
            pltpu.make_async_copy(onesv, accc.at[deff.at[k]], s_sc).wait()
            fire_rows(b ^ 1, k)

    def pair(p, carry):
        do_slot(2 * p, 0)
        do_slot(2 * p + 1, 1)
        return carry

    fire_idx(0, 0)
    wait_idx(0, 0)
    fire_st(0)
    for k in range(_K):
        fire_rows(0, k)
    lax.fori_loop(0, _NSLOT // 2, pair, None)
    wait_st(0)
    for k in range(_K):
        wait_rows(0, k)

    plsc.subcore_barrier()

    pltpu.sync_copy(acc.at[pl.ds(sid * _ZROWS, _ZROWS)],
                    p_out.at[pl.ds(cid * _NROWS + sid * _ZROWS, _ZROWS)])
    pltpu.sync_copy(accc.at[pl.ds(sid * _ZROWS, _ZROWS)],
                    c_out.at[pl.ds(cid * _NROWS + sid * _ZROWS, _ZROWS)])


_sc_call = functools.partial(
    pl.kernel,
    out_type=[
        jax.ShapeDtypeStruct((2 * _NROWS, D_FEAT), jnp.float32),
        jax.ShapeDtypeStruct((2 * _NROWS,), jnp.float32),
    ],
    mesh=plsc.VectorSubcoreMesh(core_axis_name="c", subcore_axis_name="s"),
    scratch_types=[
        pltpu.VMEM_SHARED((_NROWS, D_FEAT), jnp.float32),
        pltpu.VMEM_SHARED((_NROWS,), jnp.float32),
        pltpu.VMEM((_K, _B), jnp.int32),
        pltpu.VMEM((_K, _B), jnp.int32),
        pltpu.VMEM((_K, _B), jnp.int32),
        pltpu.VMEM((_K, _B), jnp.int32),
        pltpu.VMEM((_K, _B), jnp.int32),
        pltpu.VMEM((_K, _B), jnp.int32),
        pltpu.VMEM((_K, _B), jnp.int32),
        pltpu.VMEM((_K, _B), jnp.int32),
        pltpu.VMEM((_K, _B), jnp.int32),
        pltpu.VMEM((_K * _B, D_FEAT), jnp.float32),
        pltpu.VMEM((_B,), jnp.float32),
        pltpu.VMEM((16, D_FEAT), jnp.float32),
        pltpu.VMEM((_ZROWS,), jnp.float32),
        pltpu.SemaphoreType.DMA,
        pltpu.SemaphoreType.DMA,
        pltpu.SemaphoreType.DMA,
        pltpu.SemaphoreType.DMA,
        pltpu.SemaphoreType.DMA,
        pltpu.SemaphoreType.DMA,
    ],
)(_sc_body)


def _combine_body(x_ref, p0_ref, p1_ref, c0_ref, c1_ref, o_ref):
    cnt = c0_ref[0, 0, :] + c1_ref[0, 0, :]
    s = p0_ref[...] + p1_ref[...]
    o_ref[...] = x_ref[...] + s / jnp.clip(cnt, 1.0, None)[:, None]


_R = 1000


def _combine(x, p0, p1, c0, c1):
    return pl.pallas_call(
        _combine_body,
        grid=(N_NODES // _R,),
        in_specs=[
            pl.BlockSpec((_R, D_FEAT), lambda i: (i, 0)),
            pl.BlockSpec((_R, D_FEAT), lambda i: (i, 0)),
            pl.BlockSpec((_R, D_FEAT), lambda i: (i, 0)),
            pl.BlockSpec((1, 1, _R), lambda i: (i, 0, 0)),
            pl.BlockSpec((1, 1, _R), lambda i: (i, 0, 0)),
        ],
        out_specs=pl.BlockSpec((_R, D_FEAT), lambda i: (i, 0)),
        out_shape=jax.ShapeDtypeStruct((N_NODES, D_FEAT), jnp.float32),
    )(x, p0, p1, c0, c1)


@jax.jit
def kernel(x, edge_index, edge_time, seed_time):
    pad = _EROWS * _B - N_EDGES
    spread = jnp.arange(pad, dtype=jnp.int32) % N_NODES
    src = jnp.concatenate([edge_index[0], spread]).reshape(_EROWS, _B)
    dst = jnp.concatenate([edge_index[1], spread]).reshape(_EROWS, _B)
    et = jnp.concatenate(
        [edge_time, jnp.full((pad,), 2 ** 30, jnp.int32)]).reshape(_EROWS, _B)
    pr, cr = _sc_call(x, src, dst, et, seed_time)
    p0 = pr[:N_NODES]
    p1 = pr[_NROWS:_NROWS + N_NODES]
    c0 = cr[:N_NODES].reshape(N_NODES // _R, 1, _R)
    c1 = cr[_NROWS:_NROWS + N_NODES].reshape(N_NODES // _R, 1, _R)
    return _combine(x, p0, p1, c0, c1)
